# baseline (speedup 1.0000x reference)
import functools

import jax
import jax.numpy as jnp
import numpy as np
from jax import lax
from jax.experimental import pallas as pl
from jax.experimental.pallas import tpu as pltpu
from jax.experimental.pallas import tpu_sc as plsc

_NC = 2
_NS = 16
_NW = _NC * _NS

_SC_PARAMS = pltpu.CompilerParams(
    needs_layout_passes=False, use_tc_tiling_on_sc=False)

_GDN = lax.GatherDimensionNumbers(
    offset_dims=(), collapsed_slice_dims=(0,), start_index_map=(0,))


def _lane_shuffle(v, idx):
    return lax.gather(v, idx[:, None], dimension_numbers=_GDN,
                      slice_sizes=(1,),
                      mode=lax.GatherScatterMode.PROMISE_IN_BOUNDS)


def _lane_allsum(v):
    lanes = lax.iota(jnp.int32, 16)
    for sh in (1, 2, 4, 8):
        v = v + _lane_shuffle(v, lanes ^ sh)
    return v


def _rsqrt_newton(s):
    i = lax.bitcast_convert_type(s, jnp.int32)
    i = jnp.int32(0x5F3759DF) - lax.shift_right_logical(i, 1)
    y = lax.bitcast_convert_type(i, jnp.float32)
    h = s * 0.5
    for _ in range(2):
        y = y * (1.5 - h * y * y)
    return y


def _deinterleave_perm(D):
    perm = np.empty(D, np.int32)
    half = D // 2
    for k in range(D // 32):
        perm[32 * k:32 * k + 16] = 16 * k + np.arange(16)
        perm[32 * k + 16:32 * k + 32] = half + 16 * k + np.arange(16)
    return perm


@functools.lru_cache(maxsize=None)
def _make_scale_kernel(V, D):
    PV = V // _NW
    RC = 125
    NCH = -(-PV // RC) + (1 if (-(-PV // RC)) % 2 else 0)
    KD = D // 16
    DW = D // 2
    mesh = plsc.VectorSubcoreMesh(core_axis_name="c", subcore_axis_name="s")

    @functools.partial(
        pl.kernel,
        mesh=mesh,
        compiler_params=_SC_PARAMS,
        out_type=jax.ShapeDtypeStruct((V, DW), jnp.int32),
        scratch_types=[
            pltpu.VMEM((RC, D), jnp.float32),
            pltpu.VMEM((RC, D), jnp.float32),
            pltpu.VMEM((RC, DW), jnp.int32),
            pltpu.SemaphoreType.DMA,
            pltpu.SemaphoreType.DMA,
        ],
    )
    def scale(tab_hbm, out_hbm, in_v0, in_v1, out_v, sem0, sem1):
        wid = lax.axis_index("s") * _NC + lax.axis_index("c")
        row0 = wid * PV

        def chunk_row0(ci):
            return row0 + jnp.minimum(ci * RC, PV - RC)

        def start_fetch(ci, in_v, sem):
            pltpu.async_copy(tab_hbm.at[pl.ds(chunk_row0(ci), RC)], in_v, sem)

        def wait_fetch(in_v, sem):
            pltpu.make_async_copy(
                tab_hbm.at[pl.ds(0, RC)], in_v, sem).wait()

        def compute_chunk(ci, in_v):
            def row_body(i, carry):
                vs = [in_v[i, pl.ds(16 * k, 16)] for k in range(KD)]
                ss = vs[0] * vs[0]
                for k in range(1, KD):
                    ss = ss + vs[k] * vs[k]
                sb = _lane_allsum(ss)
                sc = jnp.minimum(_rsqrt_newton(sb), 1.0)
                sv = [v * sc for v in vs]
                for k in range(KD // 2):
                    pk = plsc.pack(sv[k], sv[k + KD // 2],
                                   format=plsc.PackFormat.INTERLEAVED)
                    out_v[i, pl.ds(16 * k, 16)] = plsc.bitcast(pk, jnp.int32)
                return carry

            lax.fori_loop(0, RC, row_body, 0)
            pltpu.sync_copy(out_v, out_hbm.at[pl.ds(chunk_row0(ci), RC)])

        start_fetch(0, in_v0, sem0)

        def pair_body(p, carry):
            ci0 = 2 * p
            wait_fetch(in_v0, sem0)
            start_fetch(ci0 + 1, in_v1, sem1)
            compute_chunk(ci0, in_v0)
            wait_fetch(in_v1, sem1)

            @pl.when(p + 1 < NCH // 2)
            def _():
                start_fetch(ci0 + 2, in_v0, sem0)

            compute_chunk(ci0 + 1, in_v1)
            return carry

        lax.fori_loop(0, NCH // 2, pair_body, 0)

    return scale


@functools.lru_cache(maxsize=None)
def _make_pool_kernel(B, L, D, V):
    CB = 32
    RPC = CB * L
    PW = B // _NW
    NCH = PW // CB
    KD = D // 32
    DW = D // 2
    mesh = plsc.VectorSubcoreMesh(core_axis_name="c", subcore_axis_name="s")

    @functools.partial(
        pl.kernel,
        mesh=mesh,
        compiler_params=_SC_PARAMS,
        out_type=jax.ShapeDtypeStruct((B, D), jnp.float32),
        scratch_types=[
            pltpu.VMEM((RPC,), jnp.int32),
            pltpu.VMEM((RPC,), jnp.int32),
            pltpu.VMEM((RPC, DW), jnp.int32),
            pltpu.VMEM((RPC, DW), jnp.int32),
            pltpu.VMEM((CB, D), jnp.float32),
            pltpu.SemaphoreType.DMA,
            pltpu.SemaphoreType.DMA,
        ],
    )
    def pool(x_hbm, table_hbm, out_hbm, idx_v0, idx_v1,
             rows_v0, rows_v1, pooled_v, sem0, sem1):
        wid = lax.axis_index("s") * _NC + lax.axis_index("c")
        base_b0 = wid * PW

        def start_fetch(ci, idx_v, rows_v, sem):
            base_r = (base_b0 + ci * CB) * L
            pltpu.sync_copy(x_hbm.at[pl.ds(base_r, RPC)], idx_v)
            pltpu.async_copy(table_hbm.at[idx_v], rows_v, sem)

        def wait_fetch(idx_v, rows_v, sem):
            pltpu.make_async_copy(table_hbm.at[idx_v], rows_v, sem).wait()

        def compute_chunk(ci, rows_v):
            def batch_body(j, carry):
                r0 = j * L
                acca = [jnp.zeros((16,), jnp.float32)] * KD
                accb = [jnp.zeros((16,), jnp.float32)] * KD
                for l in range(L):
                    r = r0 + l
                    for k in range(KD):
                        v = rows_v[r, pl.ds(16 * k, 16)]
                        vbf = plsc.bitcast(v, jnp.bfloat16)
                        a, b = plsc.unpack(vbf, format=plsc.PackFormat.INTERLEAVED)
                        acca[k] = acca[k] + a
                        accb[k] = accb[k] + b
                inv = jnp.float32(1.0 / L)
                for k in range(KD):
                    pooled_v[j, pl.ds(32 * k, 16)] = acca[k] * inv
                    pooled_v[j, pl.ds(32 * k + 16, 16)] = accb[k] * inv
                return carry

            lax.fori_loop(0, CB, batch_body, 0)
            pltpu.sync_copy(pooled_v, out_hbm.at[pl.ds(base_b0 + ci * CB, CB)])

        start_fetch(0, idx_v0, rows_v0, sem0)

        def pair_body(p, carry):
            ci0 = 2 * p
            wait_fetch(idx_v0, rows_v0, sem0)
            start_fetch(ci0 + 1, idx_v1, rows_v1, sem1)
            compute_chunk(ci0, rows_v0)
            wait_fetch(idx_v1, rows_v1, sem1)

            @pl.when(p + 1 < NCH // 2)
            def _():
                start_fetch(ci0 + 2, idx_v0, rows_v0, sem0)

            compute_chunk(ci0 + 1, rows_v1)
            return carry

        lax.fori_loop(0, NCH // 2, pair_body, 0)

    return pool


def _mlp_body(xe_ref, w1_ref, b1_ref, w2_ref, b2_ref, fc1_ref, fc2_ref, pred_ref):
    x = xe_ref[...]
    h = lax.dot_general(x, w1_ref[...], (((1,), (1,)), ((), ())),
                        preferred_element_type=jnp.float32)
    h = jnp.maximum(h + b1_ref[...], 0.0)
    fc1_ref[...] = h
    z = jnp.sum(h * w2_ref[...], axis=1, keepdims=True) + b2_ref[...]
    fc2_ref[...] = z
    pred_ref[...] = 1.0 / (1.0 + jnp.exp(-z))


@functools.lru_cache(maxsize=None)
def _make_mlp(B, D, H, BT):
    grid = (B // BT,)
    return pl.pallas_call(
        _mlp_body,
        grid=grid,
        in_specs=[
            pl.BlockSpec((BT, D), lambda i: (i, 0)),
            pl.BlockSpec((H, D), lambda i: (0, 0)),
            pl.BlockSpec((1, H), lambda i: (0, 0)),
            pl.BlockSpec((1, H), lambda i: (0, 0)),
            pl.BlockSpec((1, 1), lambda i: (0, 0)),
        ],
        out_specs=[
            pl.BlockSpec((BT, H), lambda i: (i, 0)),
            pl.BlockSpec((BT, 1), lambda i: (i, 0)),
            pl.BlockSpec((BT, 1), lambda i: (i, 0)),
        ],
        out_shape=[
            jax.ShapeDtypeStruct((B, H), jnp.float32),
            jax.ShapeDtypeStruct((B, 1), jnp.float32),
            jax.ShapeDtypeStruct((B, 1), jnp.float32),
        ],
    )


def kernel(x, table, W1, b1, W2, b2):
    B, L = x.shape
    V, D = table.shape
    H = W1.shape[0]
    x_flat = x.reshape(B * L).astype(jnp.int32)
    scaled_tab = _make_scale_kernel(V, D)(table)
    x_embed = _make_pool_kernel(B, L, D, V)(x_flat, scaled_tab)
    W1p = W1[:, _deinterleave_perm(D)]
    fc1, fc2, pred = _make_mlp(B, D, H, 1024)(
        x_embed, W1p, b1.reshape(1, H), W2, b2.reshape(1, 1))
    return fc1, fc2, pred


# speedup vs baseline: 1.0045x; 1.0045x over previous
import functools

import jax
import jax.numpy as jnp
import numpy as np
from jax import lax
from jax.experimental import pallas as pl
from jax.experimental.pallas import tpu as pltpu
from jax.experimental.pallas import tpu_sc as plsc

_NC = 2
_NS = 16
_NW = _NC * _NS

_SC_PARAMS = pltpu.CompilerParams(
    needs_layout_passes=False, use_tc_tiling_on_sc=False)

_GDN = lax.GatherDimensionNumbers(
    offset_dims=(), collapsed_slice_dims=(0,), start_index_map=(0,))


def _lane_shuffle(v, idx):
    return lax.gather(v, idx[:, None], dimension_numbers=_GDN,
                      slice_sizes=(1,),
                      mode=lax.GatherScatterMode.PROMISE_IN_BOUNDS)


def _lane_allsum(v):
    lanes = lax.iota(jnp.int32, 16)
    for sh in (1, 2, 4, 8):
        v = v + _lane_shuffle(v, lanes ^ sh)
    return v


def _rsqrt_newton(s):
    i = lax.bitcast_convert_type(s, jnp.int32)
    i = jnp.int32(0x5F3759DF) - lax.shift_right_logical(i, 1)
    y = lax.bitcast_convert_type(i, jnp.float32)
    h = s * 0.5
    for _ in range(2):
        y = y * (1.5 - h * y * y)
    return y


def _deinterleave_perm(D):
    perm = np.empty(D, np.int32)
    half = D // 2
    for k in range(D // 32):
        perm[32 * k:32 * k + 16] = 16 * k + np.arange(16)
        perm[32 * k + 16:32 * k + 32] = half + 16 * k + np.arange(16)
    return perm


@functools.lru_cache(maxsize=None)
def _make_scale_kernel(V, D):
    PV = V // _NW
    RC = 125
    NCH = -(-PV // RC) + (1 if (-(-PV // RC)) % 2 else 0)
    KD = D // 16
    DW = D // 2
    mesh = plsc.VectorSubcoreMesh(core_axis_name="c", subcore_axis_name="s")

    @functools.partial(
        pl.kernel,
        mesh=mesh,
        compiler_params=_SC_PARAMS,
        out_type=jax.ShapeDtypeStruct((V, DW), jnp.int32),
        scratch_types=[
            pltpu.VMEM((RC, D), jnp.float32),
            pltpu.VMEM((RC, D), jnp.float32),
            pltpu.VMEM((RC, DW), jnp.int32),
            pltpu.SemaphoreType.DMA,
            pltpu.SemaphoreType.DMA,
        ],
    )
    def scale(tab_hbm, out_hbm, in_v0, in_v1, out_v, sem0, sem1):
        wid = lax.axis_index("s") * _NC + lax.axis_index("c")
        row0 = wid * PV

        def chunk_row0(ci):
            return row0 + jnp.minimum(ci * RC, PV - RC)

        def start_fetch(ci, in_v, sem):
            pltpu.async_copy(tab_hbm.at[pl.ds(chunk_row0(ci), RC)], in_v, sem)

        def wait_fetch(in_v, sem):
            pltpu.make_async_copy(
                tab_hbm.at[pl.ds(0, RC)], in_v, sem).wait()

        def compute_chunk(ci, in_v):
            UNROLL = 5

            def row_body(g, carry):
                for u in range(UNROLL):
                    i = g * UNROLL + u
                    vs = [in_v[i, pl.ds(16 * k, 16)] for k in range(KD)]
                    ss = vs[0] * vs[0]
                    for k in range(1, KD):
                        ss = ss + vs[k] * vs[k]
                    sb = _lane_allsum(ss)
                    sc = jnp.minimum(_rsqrt_newton(sb), 1.0)
                    sv = [v * sc for v in vs]
                    for k in range(KD // 2):
                        pk = plsc.pack(sv[k], sv[k + KD // 2],
                                       format=plsc.PackFormat.INTERLEAVED)
                        out_v[i, pl.ds(16 * k, 16)] = plsc.bitcast(pk, jnp.int32)
                return carry

            lax.fori_loop(0, RC // UNROLL, row_body, 0)
            pltpu.sync_copy(out_v, out_hbm.at[pl.ds(chunk_row0(ci), RC)])

        start_fetch(0, in_v0, sem0)

        def pair_body(p, carry):
            ci0 = 2 * p
            wait_fetch(in_v0, sem0)
            start_fetch(ci0 + 1, in_v1, sem1)
            compute_chunk(ci0, in_v0)
            wait_fetch(in_v1, sem1)

            @pl.when(p + 1 < NCH // 2)
            def _():
                start_fetch(ci0 + 2, in_v0, sem0)

            compute_chunk(ci0 + 1, in_v1)
            return carry

        lax.fori_loop(0, NCH // 2, pair_body, 0)

    return scale


@functools.lru_cache(maxsize=None)
def _make_pool_kernel(B, L, D, V):
    CB = 32
    RPC = CB * L
    PW = B // _NW
    NCH = PW // CB
    KD = D // 32
    DW = D // 2
    mesh = plsc.VectorSubcoreMesh(core_axis_name="c", subcore_axis_name="s")

    @functools.partial(
        pl.kernel,
        mesh=mesh,
        compiler_params=_SC_PARAMS,
        out_type=jax.ShapeDtypeStruct((B, D), jnp.float32),
        scratch_types=[
            pltpu.VMEM((RPC,), jnp.int32),
            pltpu.VMEM((RPC,), jnp.int32),
            pltpu.VMEM((RPC, DW), jnp.int32),
            pltpu.VMEM((RPC, DW), jnp.int32),
            pltpu.VMEM((CB, D), jnp.float32),
            pltpu.SemaphoreType.DMA,
            pltpu.SemaphoreType.DMA,
        ],
    )
    def pool(x_hbm, table_hbm, out_hbm, idx_v0, idx_v1,
             rows_v0, rows_v1, pooled_v, sem0, sem1):
        wid = lax.axis_index("s") * _NC + lax.axis_index("c")
        base_b0 = wid * PW

        def start_fetch(ci, idx_v, rows_v, sem):
            base_r = (base_b0 + ci * CB) * L
            pltpu.sync_copy(x_hbm.at[pl.ds(base_r, RPC)], idx_v)
            pltpu.async_copy(table_hbm.at[idx_v], rows_v, sem)

        def wait_fetch(idx_v, rows_v, sem):
            pltpu.make_async_copy(table_hbm.at[idx_v], rows_v, sem).wait()

        def compute_chunk(ci, rows_v):
            def batch_body(j, carry):
                r0 = j * L
                acca = [jnp.zeros((16,), jnp.float32)] * KD
                accb = [jnp.zeros((16,), jnp.float32)] * KD
                for l in range(L):
                    r = r0 + l
                    for k in range(KD):
                        v = rows_v[r, pl.ds(16 * k, 16)]
                        vbf = plsc.bitcast(v, jnp.bfloat16)
                        a, b = plsc.unpack(vbf, format=plsc.PackFormat.INTERLEAVED)
                        acca[k] = acca[k] + a
                        accb[k] = accb[k] + b
                inv = jnp.float32(1.0 / L)
                for k in range(KD):
                    pooled_v[j, pl.ds(32 * k, 16)] = acca[k] * inv
                    pooled_v[j, pl.ds(32 * k + 16, 16)] = accb[k] * inv
                return carry

            lax.fori_loop(0, CB, batch_body, 0)
            pltpu.sync_copy(pooled_v, out_hbm.at[pl.ds(base_b0 + ci * CB, CB)])

        start_fetch(0, idx_v0, rows_v0, sem0)

        def pair_body(p, carry):
            ci0 = 2 * p
            wait_fetch(idx_v0, rows_v0, sem0)
            start_fetch(ci0 + 1, idx_v1, rows_v1, sem1)
            compute_chunk(ci0, rows_v0)
            wait_fetch(idx_v1, rows_v1, sem1)

            @pl.when(p + 1 < NCH // 2)
            def _():
                start_fetch(ci0 + 2, idx_v0, rows_v0, sem0)

            compute_chunk(ci0 + 1, rows_v1)
            return carry

        lax.fori_loop(0, NCH // 2, pair_body, 0)

    return pool


def _mlp_body(xe_ref, w1_ref, b1_ref, w2_ref, b2_ref, fc1_ref, fc2_ref, pred_ref):
    x = xe_ref[...]
    h = lax.dot_general(x, w1_ref[...], (((1,), (1,)), ((), ())),
                        preferred_element_type=jnp.float32)
    h = jnp.maximum(h + b1_ref[...], 0.0)
    fc1_ref[...] = h
    z = jnp.sum(h * w2_ref[...], axis=1, keepdims=True) + b2_ref[...]
    fc2_ref[...] = z
    pred_ref[...] = 1.0 / (1.0 + jnp.exp(-z))


@functools.lru_cache(maxsize=None)
def _make_mlp(B, D, H, BT):
    grid = (B // BT,)
    return pl.pallas_call(
        _mlp_body,
        grid=grid,
        in_specs=[
            pl.BlockSpec((BT, D), lambda i: (i, 0)),
            pl.BlockSpec((H, D), lambda i: (0, 0)),
            pl.BlockSpec((1, H), lambda i: (0, 0)),
            pl.BlockSpec((1, H), lambda i: (0, 0)),
            pl.BlockSpec((1, 1), lambda i: (0, 0)),
        ],
        out_specs=[
            pl.BlockSpec((BT, H), lambda i: (i, 0)),
            pl.BlockSpec((BT, 1), lambda i: (i, 0)),
            pl.BlockSpec((BT, 1), lambda i: (i, 0)),
        ],
        out_shape=[
            jax.ShapeDtypeStruct((B, H), jnp.float32),
            jax.ShapeDtypeStruct((B, 1), jnp.float32),
            jax.ShapeDtypeStruct((B, 1), jnp.float32),
        ],
    )


def kernel(x, table, W1, b1, W2, b2):
    B, L = x.shape
    V, D = table.shape
    H = W1.shape[0]
    x_flat = x.reshape(B * L).astype(jnp.int32)
    scaled_tab = _make_scale_kernel(V, D)(table)
    x_embed = _make_pool_kernel(B, L, D, V)(x_flat, scaled_tab)
    W1p = W1[:, _deinterleave_perm(D)]
    fc1, fc2, pred = _make_mlp(B, D, H, 1024)(
        x_embed, W1p, b1.reshape(1, H), W2, b2.reshape(1, 1))
    return fc1, fc2, pred


# speedup vs baseline: 1.0079x; 1.0034x over previous
import functools

import jax
import jax.numpy as jnp
import numpy as np
from jax import lax
from jax.experimental import pallas as pl
from jax.experimental.pallas import tpu as pltpu
from jax.experimental.pallas import tpu_sc as plsc

_NC = 2
_NS = 16
_NW = _NC * _NS

_SC_PARAMS = pltpu.CompilerParams(
    needs_layout_passes=False, use_tc_tiling_on_sc=False)

_GDN = lax.GatherDimensionNumbers(
    offset_dims=(), collapsed_slice_dims=(0,), start_index_map=(0,))


def _lane_shuffle(v, idx):
    return lax.gather(v, idx[:, None], dimension_numbers=_GDN,
                      slice_sizes=(1,),
                      mode=lax.GatherScatterMode.PROMISE_IN_BOUNDS)


def _lane_allsum(v):
    lanes = lax.iota(jnp.int32, 16)
    for sh in (1, 2, 4, 8):
        v = v + _lane_shuffle(v, lanes ^ sh)
    return v


def _rsqrt_newton(s):
    i = lax.bitcast_convert_type(s, jnp.int32)
    i = jnp.int32(0x5F3759DF) - lax.shift_right_logical(i, 1)
    y = lax.bitcast_convert_type(i, jnp.float32)
    h = s * 0.5
    for _ in range(2):
        y = y * (1.5 - h * y * y)
    return y


def _deinterleave_perm(D):
    perm = np.empty(D, np.int32)
    half = D // 2
    for k in range(D // 32):
        perm[32 * k:32 * k + 16] = 16 * k + np.arange(16)
        perm[32 * k + 16:32 * k + 32] = half + 16 * k + np.arange(16)
    return perm


@functools.lru_cache(maxsize=None)
def _make_scale_kernel(V, D):
    PV = V // _NW
    RC = 128
    NCH = -(-PV // RC) + (1 if (-(-PV // RC)) % 2 else 0)
    KD = D // 16
    DW = D // 2
    mesh = plsc.VectorSubcoreMesh(core_axis_name="c", subcore_axis_name="s")

    @functools.partial(
        pl.kernel,
        mesh=mesh,
        compiler_params=_SC_PARAMS,
        out_type=jax.ShapeDtypeStruct((V, DW), jnp.int32),
        scratch_types=[
            pltpu.VMEM((RC, D), jnp.float32),
            pltpu.VMEM((RC, D), jnp.float32),
            pltpu.VMEM((RC, DW), jnp.int32),
            pltpu.SemaphoreType.DMA,
            pltpu.SemaphoreType.DMA,
        ],
    )
    def scale(tab_hbm, out_hbm, in_v0, in_v1, out_v, sem0, sem1):
        wid = lax.axis_index("s") * _NC + lax.axis_index("c")
        row0 = wid * PV

        def chunk_row0(ci):
            return row0 + jnp.minimum(ci * RC, PV - RC)

        def start_fetch(ci, in_v, sem):
            pltpu.async_copy(tab_hbm.at[pl.ds(chunk_row0(ci), RC)], in_v, sem)

        def wait_fetch(in_v, sem):
            pltpu.make_async_copy(
                tab_hbm.at[pl.ds(0, RC)], in_v, sem).wait()

        def compute_chunk(ci, in_v):
            UNROLL = 4

            def row_body(g, carry):
                for u in range(UNROLL):
                    i = g * UNROLL + u
                    vs = [in_v[i, pl.ds(16 * k, 16)] for k in range(KD)]
                    sq = [v * v for v in vs]
                    while len(sq) > 1:
                        sq = [sq[m] + sq[m + 1] for m in range(0, len(sq), 2)]
                    sb = _lane_allsum(sq[0])
                    sc = jnp.minimum(_rsqrt_newton(sb), 1.0)
                    sv = [v * sc for v in vs]
                    for k in range(KD // 2):
                        pk = plsc.pack(sv[k], sv[k + KD // 2],
                                       format=plsc.PackFormat.INTERLEAVED)
                        out_v[i, pl.ds(16 * k, 16)] = plsc.bitcast(pk, jnp.int32)
                return carry

            lax.fori_loop(0, RC // UNROLL, row_body, 0)
            pltpu.sync_copy(out_v, out_hbm.at[pl.ds(chunk_row0(ci), RC)])

        start_fetch(0, in_v0, sem0)

        def pair_body(p, carry):
            ci0 = 2 * p
            wait_fetch(in_v0, sem0)
            start_fetch(ci0 + 1, in_v1, sem1)
            compute_chunk(ci0, in_v0)
            wait_fetch(in_v1, sem1)

            @pl.when(p + 1 < NCH // 2)
            def _():
                start_fetch(ci0 + 2, in_v0, sem0)

            compute_chunk(ci0 + 1, in_v1)
            return carry

        lax.fori_loop(0, NCH // 2, pair_body, 0)

    return scale


@functools.lru_cache(maxsize=None)
def _make_pool_kernel(B, L, D, V):
    CB = 32
    RPC = CB * L
    PW = B // _NW
    NCH = PW // CB
    KD = D // 32
    DW = D // 2
    mesh = plsc.VectorSubcoreMesh(core_axis_name="c", subcore_axis_name="s")

    @functools.partial(
        pl.kernel,
        mesh=mesh,
        compiler_params=_SC_PARAMS,
        out_type=jax.ShapeDtypeStruct((B, D), jnp.float32),
        scratch_types=[
            pltpu.VMEM((RPC,), jnp.int32),
            pltpu.VMEM((RPC,), jnp.int32),
            pltpu.VMEM((RPC, DW), jnp.int32),
            pltpu.VMEM((RPC, DW), jnp.int32),
            pltpu.VMEM((CB, D), jnp.float32),
            pltpu.SemaphoreType.DMA,
            pltpu.SemaphoreType.DMA,
        ],
    )
    def pool(x_hbm, table_hbm, out_hbm, idx_v0, idx_v1,
             rows_v0, rows_v1, pooled_v, sem0, sem1):
        wid = lax.axis_index("s") * _NC + lax.axis_index("c")
        base_b0 = wid * PW

        def start_fetch(ci, idx_v, rows_v, sem):
            base_r = (base_b0 + ci * CB) * L
            pltpu.sync_copy(x_hbm.at[pl.ds(base_r, RPC)], idx_v)
            pltpu.async_copy(table_hbm.at[idx_v], rows_v, sem)

        def wait_fetch(idx_v, rows_v, sem):
            pltpu.make_async_copy(table_hbm.at[idx_v], rows_v, sem).wait()

        def compute_chunk(ci, rows_v):
            def batch_body(j, carry):
                r0 = j * L
                acca = [jnp.zeros((16,), jnp.float32)] * KD
                accb = [jnp.zeros((16,), jnp.float32)] * KD
                for l in range(L):
                    r = r0 + l
                    for k in range(KD):
                        v = rows_v[r, pl.ds(16 * k, 16)]
                        vbf = plsc.bitcast(v, jnp.bfloat16)
                        a, b = plsc.unpack(vbf, format=plsc.PackFormat.INTERLEAVED)
                        acca[k] = acca[k] + a
                        accb[k] = accb[k] + b
                inv = jnp.float32(1.0 / L)
                for k in range(KD):
                    pooled_v[j, pl.ds(32 * k, 16)] = acca[k] * inv
                    pooled_v[j, pl.ds(32 * k + 16, 16)] = accb[k] * inv
                return carry

            lax.fori_loop(0, CB, batch_body, 0)
            pltpu.sync_copy(pooled_v, out_hbm.at[pl.ds(base_b0 + ci * CB, CB)])

        start_fetch(0, idx_v0, rows_v0, sem0)

        def pair_body(p, carry):
            ci0 = 2 * p
            wait_fetch(idx_v0, rows_v0, sem0)
            start_fetch(ci0 + 1, idx_v1, rows_v1, sem1)
            compute_chunk(ci0, rows_v0)
            wait_fetch(idx_v1, rows_v1, sem1)

            @pl.when(p + 1 < NCH // 2)
            def _():
                start_fetch(ci0 + 2, idx_v0, rows_v0, sem0)

            compute_chunk(ci0 + 1, rows_v1)
            return carry

        lax.fori_loop(0, NCH // 2, pair_body, 0)

    return pool


def _mlp_body(xe_ref, w1_ref, b1_ref, w2_ref, b2_ref, fc1_ref, fc2_ref, pred_ref):
    x = xe_ref[...]
    h = lax.dot_general(x, w1_ref[...], (((1,), (1,)), ((), ())),
                        preferred_element_type=jnp.float32)
    h = jnp.maximum(h + b1_ref[...], 0.0)
    fc1_ref[...] = h
    z = jnp.sum(h * w2_ref[...], axis=1, keepdims=True) + b2_ref[...]
    fc2_ref[...] = z
    pred_ref[...] = 1.0 / (1.0 + jnp.exp(-z))


@functools.lru_cache(maxsize=None)
def _make_mlp(B, D, H, BT):
    grid = (B // BT,)
    return pl.pallas_call(
        _mlp_body,
        grid=grid,
        in_specs=[
            pl.BlockSpec((BT, D), lambda i: (i, 0)),
            pl.BlockSpec((H, D), lambda i: (0, 0)),
            pl.BlockSpec((1, H), lambda i: (0, 0)),
            pl.BlockSpec((1, H), lambda i: (0, 0)),
            pl.BlockSpec((1, 1), lambda i: (0, 0)),
        ],
        out_specs=[
            pl.BlockSpec((BT, H), lambda i: (i, 0)),
            pl.BlockSpec((BT, 1), lambda i: (i, 0)),
            pl.BlockSpec((BT, 1), lambda i: (i, 0)),
        ],
        out_shape=[
            jax.ShapeDtypeStruct((B, H), jnp.float32),
            jax.ShapeDtypeStruct((B, 1), jnp.float32),
            jax.ShapeDtypeStruct((B, 1), jnp.float32),
        ],
    )


def kernel(x, table, W1, b1, W2, b2):
    B, L = x.shape
    V, D = table.shape
    H = W1.shape[0]
    x_flat = x.reshape(B * L).astype(jnp.int32)
    scaled_tab = _make_scale_kernel(V, D)(table)
    x_embed = _make_pool_kernel(B, L, D, V)(x_flat, scaled_tab)
    W1p = W1[:, _deinterleave_perm(D)]
    fc1, fc2, pred = _make_mlp(B, D, H, 1024)(
        x_embed, W1p, b1.reshape(1, H), W2, b2.reshape(1, 1))
    return fc1, fc2, pred
